# baseline (speedup 1.0000x reference)
import functools

import jax
import jax.numpy as jnp
from jax import lax
from jax.experimental import pallas as pl
from jax.experimental.pallas import tpu as pltpu
from jax.experimental.pallas import tpu_sc as plsc

_L = 16


def _build_sc_kernel(B, T, E, V, scale):
    info = plsc.get_sparse_core_info()
    NC, NS = info.num_cores, info.num_subcores
    NW = NC * NS
    b_per_w = B // NW
    R = b_per_w * T
    CH = 128
    NCHUNK = R // CH
    NVEC = CH // _L
    OW = CH * E // 128
    E2 = 2 * E
    mesh = plsc.VectorSubcoreMesh(core_axis_name="c", subcore_axis_name="s")

    @functools.partial(
        pl.kernel,
        out_type=jax.ShapeDtypeStruct((B * T * E // 128, 128), jnp.float32),
        mesh=mesh,
        compiler_params=pltpu.CompilerParams(disable_bounds_checks=True),
        scratch_types=[
            pltpu.VMEM((T * E,), jnp.float32),
            pltpu.VMEM((E,), jnp.float32),
            pltpu.VMEM((R + _L,), jnp.int32),
            pltpu.VMEM((R + _L,), jnp.float32),
            pltpu.VMEM((NCHUNK, CH), jnp.int32),
            pltpu.VMEM((2, CH, E2), jnp.float32),
            pltpu.VMEM((2, OW, 128), jnp.float32),
            pltpu.SemaphoreType.DMA,
            pltpu.SemaphoreType.DMA,
            pltpu.SemaphoreType.DMA,
            pltpu.SemaphoreType.DMA,
        ],
    )
    def sc_kernel(ids_hbm, maskf_hbm, tpair_hbm, tlast_hbm, pos_hbm, out_hbm,
                  pos_v, tl_v, ids_v, mf_v, mid_v, rows_v, obuf_v,
                  gsem0, gsem1, wsem0, wsem1):
        gsem = (gsem0, gsem1)
        wsem = (wsem0, wsem1)
        wid = lax.axis_index("s") * NC + lax.axis_index("c")
        w0 = pl.multiple_of(wid * R, R)
        pltpu.sync_copy(pos_hbm, pos_v)
        pltpu.sync_copy(tlast_hbm, tl_v)
        pltpu.sync_copy(ids_hbm.at[pl.ds(w0, R)], ids_v.at[pl.ds(0, R)])
        pltpu.sync_copy(maskf_hbm.at[pl.ds(w0, R)], mf_v.at[pl.ds(0, R)])

        def idx_chunk(g, carry):
            def idx_vec(v, c):
                off = g * CH + v * _L
                ids16 = ids_v[pl.ds(off, _L)]
                keep = mf_v[pl.ds(off, _L)] > 0.5
                pid = lax.shift_right_logical(ids16, 1)
                pid = jnp.where(ids16 == (V - 1), 0, pid)
                mid_v[g, pl.ds(v * _L, _L)] = jnp.where(keep, pid, 0)
                return c
            return lax.fori_loop(0, NVEC, idx_vec, carry)
        lax.fori_loop(0, NCHUNK, idx_chunk, 0)

        def gather(g, p):
            def issue(v, c):
                iv = mid_v[g, pl.ds(v * _L, _L)]
                pltpu.async_copy(
                    tpair_hbm.at[iv],
                    rows_v.at[p].at[pl.ds(v * _L, _L)],
                    gsem[p])
                return c
            lax.fori_loop(0, NVEC, issue, 0)

        def gather_wait(p):
            pltpu.make_async_copy(
                tpair_hbm.at[pl.ds(0, CH)], rows_v.at[p], gsem[p]).wait()

        def compute(g, p):
            t0 = (g * CH) % T

            def body(r, c):
                i = ids_v[pl.ds(g * CH + r, _L)][0]
                m = mf_v[pl.ds(g * CH + r, _L)][0]
                keep = m > 0.5
                h = jnp.where(keep, i & 1, 0) * E
                t = t0 + r
                t = jnp.where(t >= T, t - T, t)

                @pl.when(jnp.logical_and(keep, i == V - 1))
                def _patch():
                    for q in range(E // _L):
                        rows_v[p, r, pl.ds(q * _L, _L)] = tl_v[pl.ds(q * _L, _L)]

                for q in range(E // _L):
                    vec = (rows_v[p, r, pl.ds(h + q * _L, _L)] * scale
                           + pos_v[pl.ds(t * E + q * _L, _L)] * m)
                    obuf_v[p, lax.shift_right_logical(r, 1),
                           pl.ds((r & 1) * E + q * _L, _L)] = vec
                return c
            lax.fori_loop(0, CH, body, 0)

        def writeback(g, p):
            return pltpu.async_copy(
                obuf_v.at[p],
                out_hbm.at[pl.ds(
                    pl.multiple_of((w0 + g * CH) * E // 128, 8), OW)],
                wsem[p])

        wps = [None, None]
        gather(0, 0)
        gather(1, 1)
        for g in range(NCHUNK):
            p = g & 1
            gather_wait(p)
            if g >= 2:
                wps[p].wait()
            compute(g, p)
            wps[p] = writeback(g, p)
            if g + 2 < NCHUNK:
                gather(g + 2, p)
        wps[0].wait()
        wps[1].wait()

    return sc_kernel


def kernel(item_id, padding_mask, item_table, pos_table):
    B, T = item_id.shape
    V, E = item_table.shape
    scale = float(E) ** 0.5
    ids = item_id.astype(jnp.int32).reshape(-1)
    maskf = padding_mask.astype(jnp.float32).reshape(-1)
    tpair = item_table[:V - 1].reshape((V - 1) // 2, 2 * E)
    tlast = item_table[V - 1]
    pos1 = pos_table.reshape(-1)
    sc = _build_sc_kernel(B, T, E, V, scale)
    out = sc(ids, maskf, tpair, tlast, pos1)
    return out.reshape(B, T, E)


# speedup vs baseline: 5.1762x; 5.1762x over previous
import functools

import jax
import jax.numpy as jnp
from jax import lax
from jax.experimental import pallas as pl
from jax.experimental.pallas import tpu as pltpu
from jax.experimental.pallas import tpu_sc as plsc

_L = 16


def _build_sc_kernel(B, T, E, scale):
    info = plsc.get_sparse_core_info()
    NC, NS = info.num_cores, info.num_subcores
    NW = NC * NS
    b_per_w = B // NW
    R = b_per_w * T
    BPC = 2
    CH = BPC * T
    NCHUNK = R // CH
    NVEC = CH // _L
    mesh = plsc.VectorSubcoreMesh(core_axis_name="c", subcore_axis_name="s")

    @functools.partial(
        pl.kernel,
        out_type=jax.ShapeDtypeStruct((B * T, E), jnp.float32),
        mesh=mesh,
        compiler_params=pltpu.CompilerParams(
            use_tc_tiling_on_sc=False, disable_bounds_checks=True,
            needs_layout_passes=False),
        scratch_types=[
            pltpu.VMEM((T, E), jnp.float32),
            pltpu.VMEM((R,), jnp.int32),
            pltpu.VMEM((R + _L,), jnp.float32),
            pltpu.VMEM((NCHUNK, CH), jnp.int32),
            pltpu.VMEM((2, CH, E), jnp.float32),
            pltpu.SMEM((NCHUNK,), jnp.int32),
            pltpu.SemaphoreType.DMA,
            pltpu.SemaphoreType.DMA,
            pltpu.SemaphoreType.DMA,
            pltpu.SemaphoreType.DMA,
        ],
    )
    def sc_kernel(ids_hbm, maskf_hbm, table_hbm, pos_hbm, out_hbm,
                  pos_v, ids_v, mf_v, mid_v, rows_v, wcnt_s,
                  gsem0, gsem1, wsem0, wsem1):
        gsem = (gsem0, gsem1)
        wsem = (wsem0, wsem1)
        wid = lax.axis_index("s") * NC + lax.axis_index("c")
        w0 = wid * R
        pltpu.sync_copy(pos_hbm, pos_v)
        pltpu.sync_copy(ids_hbm.at[pl.ds(w0, R)], ids_v)
        pltpu.sync_copy(maskf_hbm.at[pl.ds(w0, R)], mf_v.at[pl.ds(0, R)])

        zeros = jnp.zeros((_L,), jnp.float32)

        def zrow(r, c):
            for pp in range(2):
                for q in range(E // _L):
                    rows_v[pp, r, pl.ds(q * _L, _L)] = zeros
            return c
        lax.fori_loop(0, CH, zrow, 0)

        def idx_chunk(g, carry):
            def idx_vec(v, c):
                off = g * CH + v * _L
                keep = mf_v[pl.ds(off, _L)] > 0.5
                mid_v[g, pl.ds(v * _L, _L)] = jnp.where(
                    keep, ids_v[pl.ds(off, _L)], -1)
                return c
            return lax.fori_loop(0, NVEC, idx_vec, carry)
        lax.fori_loop(0, NCHUNK, idx_chunk, 0)

        def gather(g, p):
            pltpu.async_copy(
                table_hbm.at[plsc.Indices(mid_v.at[g], ignored_value=-1)],
                rows_v.at[p], gsem[p])

        def gather_wait(g, p):
            pltpu.make_async_copy(
                table_hbm.at[pl.ds(0, CH)], rows_v.at[p], gsem[p]).wait()

        def compute(g, p):
            for sub in range(BPC):
                def body(t, c, sub=sub):
                    r = sub * T + t
                    m = mf_v[pl.ds(g * CH + r, _L)][0]
                    sm = m * scale
                    for q in range(E // _L):
                        sl = pl.ds(q * _L, _L)
                        rows_v[p, r, sl] = (rows_v[p, r, sl] * sm
                                            + pos_v[t, sl] * m)
                    return c
                lax.fori_loop(0, T, body, 0)

        def writeback(g, p):
            return pltpu.async_copy(
                rows_v.at[p], out_hbm.at[pl.ds(w0 + g * CH, CH)], wsem[p])

        wps = [None, None]
        gather(0, 0)
        gather(1, 1)
        for g in range(NCHUNK):
            p = g & 1
            gather_wait(g, p)
            compute(g, p)
            wps[p] = writeback(g, p)
            if g + 2 < NCHUNK:
                wps[p].wait()
                gather(g + 2, p)
        wps[0].wait()
        wps[1].wait()

    return sc_kernel


def kernel(item_id, padding_mask, item_table, pos_table):
    B, T = item_id.shape
    V, E = item_table.shape
    scale = float(E) ** 0.5
    ids = item_id.astype(jnp.int32).reshape(-1)
    maskf = padding_mask.astype(jnp.float32).reshape(-1)
    sc = _build_sc_kernel(B, T, E, scale)
    out = sc(ids, maskf, item_table, pos_table)
    return out.reshape(B, T, E)
